# baseline (speedup 1.0000x reference)
import functools

import jax
import jax.numpy as jnp
from jax import lax
from jax.experimental import pallas as pl
from jax.experimental.pallas import tpu as pltpu
from jax.experimental.pallas import tpu_sc as plsc

ROWS = 16384
COLS = 26
DIM = 16
VOCAB = 2600000
ESTRIDE = 2600064
FIELD = 100000
NC = 2
NS = 16
NW = NC * NS
RPW = ROWS // NW
LANES = 16

CH = 4096
NFULL = VOCAB // CH
TAILV = NFULL * CH
TAILCH = 3072
NITEM = 2 * NFULL
NITER = (NITEM + NW - 1) // NW


def _detile_body(tt_hbm, tail_hbm, tlin_hbm, scr, tscr, isem, osem):
    wid = lax.axis_index("s") * NC + lax.axis_index("c")

    def item(k):
        k = lax.rem(k, NITEM)
        g = k // NFULL
        v0 = pl.multiple_of((k % NFULL) * CH, 128)
        g8 = pl.multiple_of(g * 8, 8)
        return g, g8, v0

    def fire_in(k, p):
        _, g8, v0 = item(k)
        pltpu.async_copy(tt_hbm.at[pl.ds(g8, 8), pl.ds(v0, CH)],
                         scr.at[p], isem)

    fire_in(wid, 0)

    def do_item(j, carry):
        p = lax.rem(j, 2)
        pltpu.make_async_copy(
            tt_hbm.at[pl.ds(0, 8), pl.ds(0, CH)], scr.at[p], isem).wait()
        fire_in(wid + NW * (j + 1), 1 - p)
        g, _, v0 = item(wid + NW * j)
        outs = [
            pltpu.async_copy(
                scr.at[p, r],
                tlin_hbm.at[pl.ds((g * 8 + r) * ESTRIDE + v0, CH)], osem)
            for r in range(8)
        ]
        for cp in outs:
            cp.wait()
        return carry

    lax.fori_loop(0, NITER, do_item, 0)
    pltpu.make_async_copy(
        tt_hbm.at[pl.ds(0, 8), pl.ds(0, CH)],
        scr.at[NITER % 2], isem).wait()

    @pl.when(wid < 2)
    def _tail_block():
        g8 = pl.multiple_of(wid * 8, 8)
        pltpu.sync_copy(
            tt_hbm.at[pl.ds(g8, 8), pl.ds(TAILV, TAILCH)],
            scr.at[0, :, pl.ds(0, TAILCH)],
        )
        for r in range(8):
            pltpu.sync_copy(
                scr.at[0, r, pl.ds(0, TAILCH)],
                tlin_hbm.at[pl.ds((wid * 8 + r) * ESTRIDE + TAILV, TAILCH)],
            )

    @pl.when(wid >= NW - DIM)
    def _tail64():
        e = wid - (NW - DIM)
        pltpu.sync_copy(tail_hbm.at[pl.ds(e * 64, 64)], tscr)
        pltpu.sync_copy(
            tscr,
            tlin_hbm.at[pl.ds(e * ESTRIDE + (VOCAB - 64), 64)],
        )


_detile_call = pl.kernel(
    _detile_body,
    out_type=jax.ShapeDtypeStruct((DIM * ESTRIDE,), jnp.float32),
    mesh=plsc.VectorSubcoreMesh(core_axis_name="c", subcore_axis_name="s"),
    scratch_types=[
        pltpu.VMEM((2, 8, CH), jnp.float32),
        pltpu.VMEM((64,), jnp.float32),
        pltpu.SemaphoreType.DMA,
        pltpu.SemaphoreType.DMA,
    ],
)


def _gather_body(xt_hbm, tlin_hbm, out_hbm, xcol, idx2d, rows2, gsem, osem):
    wid = lax.axis_index("s") * NC + lax.axis_index("c")
    r0 = wid * RPW

    def do_col(c, carry):
        pltpu.sync_copy(xt_hbm.at[c, pl.ds(r0, RPW)], xcol)

        def add_off(i, _):
            o = pl.multiple_of(i * LANES, LANES)
            vv = xcol[pl.ds(o, LANES)] + c * FIELD
            for e in range(DIM):
                idx2d[e, pl.ds(o, LANES)] = vv + e * ESTRIDE
            return _
        lax.fori_loop(0, RPW // LANES, add_off, 0, unroll=2)

        p = lax.rem(c, 2)

        @pl.when(c >= 2)
        def _():
            pltpu.make_async_copy(
                rows2.at[p], out_hbm.at[c, :, pl.ds(r0, RPW)], osem
            ).wait()

        cps = [
            pltpu.async_copy(tlin_hbm.at[idx2d.at[e]], rows2.at[p, e], gsem)
            for e in range(DIM)
        ]
        for cp in cps:
            cp.wait()
        pltpu.async_copy(rows2.at[p], out_hbm.at[c, :, pl.ds(r0, RPW)], osem)
        return carry

    lax.fori_loop(0, COLS, do_col, 0)

    for c in (COLS - 2, COLS - 1):
        pltpu.make_async_copy(
            rows2.at[c % 2], out_hbm.at[c, :, pl.ds(r0, RPW)], osem
        ).wait()


_gather_call = pl.kernel(
    _gather_body,
    out_type=jax.ShapeDtypeStruct((COLS, DIM, ROWS), jnp.float32),
    mesh=plsc.VectorSubcoreMesh(core_axis_name="c", subcore_axis_name="s"),
    compiler_params=pltpu.CompilerParams(use_tc_tiling_on_sc=False),
    scratch_types=[
        pltpu.VMEM((RPW,), jnp.int32),
        pltpu.VMEM((DIM, RPW), jnp.int32),
        pltpu.VMEM((2, DIM, RPW), jnp.float32),
        pltpu.SemaphoreType.DMA,
        pltpu.SemaphoreType.DMA,
    ],
)


@jax.jit
def kernel(x, table):
    xt = x.T.astype(jnp.int32)
    tail = jnp.swapaxes(lax.slice(table, (VOCAB - 64, 0), (VOCAB, DIM)),
                        0, 1).reshape(DIM * 64)
    tlin = _detile_call(table.T, tail)
    out = _gather_call(xt, tlin)
    return out.transpose(2, 0, 1)


# speedup vs baseline: 1.0606x; 1.0606x over previous
import functools

import jax
import jax.numpy as jnp
from jax import lax
from jax.experimental import pallas as pl
from jax.experimental.pallas import tpu as pltpu
from jax.experimental.pallas import tpu_sc as plsc

ROWS = 16384
COLS = 26
DIM = 16
VOCAB = 2600000
ESTRIDE = 2600064
FIELD = 100000
NC = 2
NS = 16
NW = NC * NS
RPW = ROWS // NW
LANES = 16

CH = 4096
NFULL = VOCAB // CH
TAILV = NFULL * CH
TAILCH = 3072
NITEM = 2 * NFULL
NITER = (NITEM + NW - 1) // NW


def _detile_body(tt_hbm, tail_hbm, tlin_hbm, scr, tscr, isem, osem):
    wid = lax.axis_index("s") * NC + lax.axis_index("c")

    def item(k):
        k = lax.rem(k, NITEM)
        g = k // NFULL
        v0 = pl.multiple_of((k % NFULL) * CH, 128)
        g8 = pl.multiple_of(g * 8, 8)
        return g, g8, v0

    def fire_in(k, p):
        _, g8, v0 = item(k)
        pltpu.async_copy(tt_hbm.at[pl.ds(g8, 8), pl.ds(v0, CH)],
                         scr.at[p], isem)

    fire_in(wid, 0)

    def do_item(j, carry):
        p = lax.rem(j, 2)
        pltpu.make_async_copy(
            tt_hbm.at[pl.ds(0, 8), pl.ds(0, CH)], scr.at[p], isem).wait()
        fire_in(wid + NW * (j + 1), 1 - p)
        g, _, v0 = item(wid + NW * j)
        outs = [
            pltpu.async_copy(
                scr.at[p, r],
                tlin_hbm.at[pl.ds((g * 8 + r) * ESTRIDE + v0, CH)], osem)
            for r in range(8)
        ]
        for cp in outs:
            cp.wait()
        return carry

    lax.fori_loop(0, NITER, do_item, 0)
    pltpu.make_async_copy(
        tt_hbm.at[pl.ds(0, 8), pl.ds(0, CH)],
        scr.at[NITER % 2], isem).wait()

    @pl.when(wid < 2)
    def _tail_block():
        g8 = pl.multiple_of(wid * 8, 8)
        pltpu.sync_copy(
            tt_hbm.at[pl.ds(g8, 8), pl.ds(TAILV, TAILCH)],
            scr.at[0, :, pl.ds(0, TAILCH)],
        )
        for r in range(8):
            pltpu.sync_copy(
                scr.at[0, r, pl.ds(0, TAILCH)],
                tlin_hbm.at[pl.ds((wid * 8 + r) * ESTRIDE + TAILV, TAILCH)],
            )

    @pl.when(wid >= NW - DIM)
    def _tail64():
        e = wid - (NW - DIM)
        pltpu.sync_copy(tail_hbm.at[pl.ds(e * 64, 64)], tscr)
        pltpu.sync_copy(
            tscr,
            tlin_hbm.at[pl.ds(e * ESTRIDE + (VOCAB - 64), 64)],
        )


_detile_call = pl.kernel(
    _detile_body,
    out_type=jax.ShapeDtypeStruct((DIM * ESTRIDE,), jnp.float32),
    mesh=plsc.VectorSubcoreMesh(core_axis_name="c", subcore_axis_name="s"),
    scratch_types=[
        pltpu.VMEM((2, 8, CH), jnp.float32),
        pltpu.VMEM((64,), jnp.float32),
        pltpu.SemaphoreType.DMA,
        pltpu.SemaphoreType.DMA,
    ],
)


def _gather_body(xt_hbm, tlin_hbm, out_hbm, xcol, idx2d, rows2, gsem, osem):
    wid = lax.axis_index("s") * NC + lax.axis_index("c")
    r0 = wid * RPW

    def drain_gathers(p):
        for e in range(DIM):
            pltpu.make_async_copy(
                tlin_hbm.at[pl.ds(0, RPW)], rows2.at[p, e], gsem).wait()

    def fire_out(c, p):
        pltpu.async_copy(rows2.at[p], out_hbm.at[c, :, pl.ds(r0, RPW)], osem)

    def drain_out(p):
        pltpu.make_async_copy(
            rows2.at[p], out_hbm.at[0, :, pl.ds(r0, RPW)], osem).wait()

    def do_col(c, carry):
        p = lax.rem(c, 2)
        pltpu.sync_copy(xt_hbm.at[c, pl.ds(r0, RPW)], xcol)

        def add_off(i, _):
            o = pl.multiple_of(i * LANES, LANES)
            vv = xcol[pl.ds(o, LANES)] + c * FIELD
            for e in range(DIM):
                idx2d[p, e, pl.ds(o, LANES)] = vv + e * ESTRIDE
            return _
        lax.fori_loop(0, RPW // LANES, add_off, 0, unroll=2)

        @pl.when(c >= 2)
        def _():
            drain_out(p)

        for e in range(DIM):
            pltpu.async_copy(
                tlin_hbm.at[idx2d.at[p, e]], rows2.at[p, e], gsem)

        @pl.when(c >= 1)
        def _():
            drain_gathers(1 - p)
            fire_out(c - 1, 1 - p)
        return carry

    lax.fori_loop(0, COLS, do_col, 0)

    last_p = (COLS - 1) % 2
    drain_gathers(last_p)
    fire_out(COLS - 1, last_p)
    drain_out(1 - last_p)
    drain_out(last_p)


_gather_call = pl.kernel(
    _gather_body,
    out_type=jax.ShapeDtypeStruct((COLS, DIM, ROWS), jnp.float32),
    mesh=plsc.VectorSubcoreMesh(core_axis_name="c", subcore_axis_name="s"),
    compiler_params=pltpu.CompilerParams(use_tc_tiling_on_sc=False),
    scratch_types=[
        pltpu.VMEM((RPW,), jnp.int32),
        pltpu.VMEM((2, DIM, RPW), jnp.int32),
        pltpu.VMEM((2, DIM, RPW), jnp.float32),
        pltpu.SemaphoreType.DMA,
        pltpu.SemaphoreType.DMA,
    ],
)


@jax.jit
def kernel(x, table):
    xt = x.T.astype(jnp.int32)
    tail = jnp.swapaxes(lax.slice(table, (VOCAB - 64, 0), (VOCAB, DIM)),
                        0, 1).reshape(DIM * 64)
    tlin = _detile_call(table.T, tail)
    out = _gather_call(xt, tlin)
    return out.transpose(2, 0, 1)
